# Initial kernel scaffold; baseline (speedup 1.0000x reference)
import jax
import jax.numpy as jnp
from jax.experimental import pallas as pl


def kernel(inputs, table):
    raise NotImplementedError("write your pallas kernel here")


# speedup vs baseline: 2.5075x; 2.5075x over previous
import functools
import math

import jax
import jax.numpy as jnp
from jax import lax
from jax.experimental import pallas as pl
from jax.experimental.pallas import tpu as pltpu
from jax.experimental.pallas import tpu_sc as plsc

D = 128
SCALE = math.sqrt(float(D))
LANES = 16
CHUNK = 128


@functools.cache
def _build(B, V):
    info = plsc.get_sparse_core_info()
    nw = info.num_cores * info.num_subcores
    b_per_w = B // nw
    num_chunks = b_per_w // CHUNK
    mesh = plsc.VectorSubcoreMesh(core_axis_name="c", subcore_axis_name="s")

    @functools.partial(
        pl.kernel,
        out_type=jax.ShapeDtypeStruct((B, D), jnp.float32),
        mesh=mesh,
        scratch_types=[
            pltpu.VMEM((b_per_w,), jnp.int32),
            pltpu.VMEM((CHUNK, D), jnp.float32),
            pltpu.SemaphoreType.DMA,
        ],
    )
    def emb_kernel(idx_hbm, table_hbm, out_hbm, idx_v, rows, gsem):
        wid = lax.axis_index("s") * info.num_cores + lax.axis_index("c")
        base = wid * b_per_w
        pltpu.sync_copy(idx_hbm.at[pl.ds(base, b_per_w)], idx_v)

        @pl.loop(0, num_chunks)
        def _chunk(g):
            off = g * CHUNK
            pltpu.async_copy(
                table_hbm.at[idx_v.at[pl.ds(off, CHUNK)]], rows, gsem
            ).wait()

            @pl.loop(0, CHUNK // LANES)
            def _grp(k):
                ivec = idx_v[pl.ds(off + k * LANES, LANES)]
                fvec = jnp.where(
                    ivec == 0, jnp.float32(0.0), jnp.float32(SCALE)
                )
                for j in range(LANES):
                    f = fvec[j]
                    for c in range(D // LANES):
                        sl = pl.ds(c * LANES, LANES)
                        rows[k * LANES + j, sl] = rows[k * LANES + j, sl] * f

            pltpu.sync_copy(rows, out_hbm.at[pl.ds(base + off, CHUNK)])

    return emb_kernel


def kernel(inputs, table):
    n, s = inputs.shape
    out = _build(n * s, table.shape[0])(inputs.reshape(-1), table)
    return out.reshape(n, s, D)


# speedup vs baseline: 2.9969x; 1.1952x over previous
import functools
import math

import jax
import jax.numpy as jnp
from jax import lax
from jax.experimental import pallas as pl
from jax.experimental.pallas import tpu as pltpu
from jax.experimental.pallas import tpu_sc as plsc

D = 128
SCALE = math.sqrt(float(D))
LANES = 16
CHUNK = 128


@functools.cache
def _build(B, V):
    info = plsc.get_sparse_core_info()
    nw = info.num_cores * info.num_subcores
    b_per_w = B // nw
    num_chunks = b_per_w // CHUNK
    mesh = plsc.VectorSubcoreMesh(core_axis_name="c", subcore_axis_name="s")

    @functools.partial(
        pl.kernel,
        out_type=jax.ShapeDtypeStruct((B, D), jnp.float32),
        mesh=mesh,
        scratch_types=[
            pltpu.VMEM((b_per_w,), jnp.int32),
            pltpu.VMEM((CHUNK, D), jnp.float32),
            pltpu.VMEM((CHUNK, D), jnp.float32),
            pltpu.SemaphoreType.DMA,
            pltpu.SemaphoreType.DMA,
        ],
    )
    def emb_kernel(idx_hbm, table_hbm, out_hbm, idx_v, rows0, rows1, s0, s1):
        wid = lax.axis_index("s") * info.num_cores + lax.axis_index("c")
        base = wid * b_per_w
        pltpu.sync_copy(idx_hbm.at[pl.ds(base, b_per_w)], idx_v)

        bufs = (rows0, rows1)
        sems = (s0, s1)

        def fire(g, slot):
            pltpu.async_copy(
                table_hbm.at[idx_v.at[pl.ds(g * CHUNK, CHUNK)]],
                bufs[slot],
                sems[slot],
            )

        def scale_and_store(g, slot):
            rows = bufs[slot]
            off = g * CHUNK
            pltpu.make_async_copy(
                table_hbm.at[idx_v.at[pl.ds(off, CHUNK)]], rows, sems[slot]
            ).wait()

            @pl.loop(0, CHUNK // LANES)
            def _grp(k):
                ivec = idx_v[pl.ds(off + k * LANES, LANES)]
                fvec = jnp.where(
                    ivec == 0, jnp.float32(0.0), jnp.float32(SCALE)
                )
                for j in range(LANES):
                    f = fvec[j]
                    for c in range(D // LANES):
                        sl = pl.ds(c * LANES, LANES)
                        rows[k * LANES + j, sl] = rows[k * LANES + j, sl] * f

            pltpu.sync_copy(rows, out_hbm.at[pl.ds(base + off, CHUNK)])

        fire(0, 0)

        @pl.loop(0, num_chunks, step=2)
        def _pair(g):
            fire(g + 1, 1)
            scale_and_store(g, 0)

            @pl.when(g + 2 < num_chunks)
            def _():
                fire(g + 2, 0)

            scale_and_store(g + 1, 1)

    return emb_kernel


def kernel(inputs, table):
    n, s = inputs.shape
    out = _build(n * s, table.shape[0])(inputs.reshape(-1), table)
    return out.reshape(n, s, D)


# speedup vs baseline: 3.0628x; 1.0220x over previous
import functools
import math

import jax
import jax.numpy as jnp
from jax import lax
from jax.experimental import pallas as pl
from jax.experimental.pallas import tpu as pltpu
from jax.experimental.pallas import tpu_sc as plsc

D = 128
SCALE = math.sqrt(float(D))
LANES = 16
CHUNK = 80
NBUF = 4


@functools.cache
def _build(B, V):
    info = plsc.get_sparse_core_info()
    nw = info.num_cores * info.num_subcores
    b_per_w = B // nw
    num_chunks = b_per_w // CHUNK
    assert num_chunks % NBUF == 0
    mesh = plsc.VectorSubcoreMesh(core_axis_name="c", subcore_axis_name="s")

    @functools.partial(
        pl.kernel,
        out_type=jax.ShapeDtypeStruct((B, D), jnp.float32),
        mesh=mesh,
        scratch_types=[
            pltpu.VMEM((b_per_w,), jnp.int32),
            [pltpu.VMEM((CHUNK, D), jnp.float32) for _ in range(NBUF)],
            [pltpu.SemaphoreType.DMA for _ in range(NBUF)],
            [pltpu.SemaphoreType.DMA for _ in range(NBUF)],
        ],
    )
    def emb_kernel(idx_hbm, table_hbm, out_hbm, idx_v, bufs, gsems, osems):
        wid = lax.axis_index("s") * info.num_cores + lax.axis_index("c")
        base = wid * b_per_w
        pltpu.sync_copy(idx_hbm.at[pl.ds(base, b_per_w)], idx_v)

        def gather(g, slot):
            return pltpu.make_async_copy(
                table_hbm.at[idx_v.at[pl.ds(g * CHUNK, CHUNK)]],
                bufs[slot],
                gsems[slot],
            )

        def out_copy(g, slot):
            return pltpu.make_async_copy(
                bufs[slot],
                out_hbm.at[pl.ds(base + g * CHUNK, CHUNK)],
                osems[slot],
            )

        def scale(g, slot):
            rows = bufs[slot]
            off = g * CHUNK

            @pl.loop(0, CHUNK // LANES)
            def _grp(k):
                ivec = idx_v[pl.ds(off + k * LANES, LANES)]
                fvec = jnp.where(
                    ivec == 0, jnp.float32(0.0), jnp.float32(SCALE)
                )
                for j in range(LANES):
                    f = fvec[j]
                    for c in range(D // LANES):
                        sl = pl.ds(c * LANES, LANES)
                        rows[k * LANES + j, sl] = rows[k * LANES + j, sl] * f

        for b in range(NBUF - 1):
            gather(b, b).start()

        @pl.loop(0, num_chunks, step=NBUF)
        def _ring(g):
            for b in range(NBUF):
                gc = g + b
                gather(gc, b).wait()
                scale(gc, b)
                out_copy(gc, b).start()
                nxt = gc + NBUF - 1
                ns = (b + NBUF - 1) % NBUF
                if b == 0:
                    @pl.when(gc == 0)
                    def _():
                        gather(nxt, ns).start()

                    @pl.when(jnp.logical_and(gc > 0, nxt < num_chunks))
                    def _():
                        out_copy(gc - 1, ns).wait()
                        gather(nxt, ns).start()
                else:
                    @pl.when(nxt < num_chunks)
                    def _():
                        out_copy(gc - 1, ns).wait()
                        gather(nxt, ns).start()

        for b in range(NBUF):
            out_copy(num_chunks - NBUF + b, b).wait()

    return emb_kernel


def kernel(inputs, table):
    n, s = inputs.shape
    out = _build(n * s, table.shape[0])(inputs.reshape(-1), table)
    return out.reshape(n, s, D)


# speedup vs baseline: 4.6803x; 1.5281x over previous
import functools
import math

import jax
import jax.numpy as jnp
from jax import lax
from jax.experimental import pallas as pl
from jax.experimental.pallas import tpu as pltpu
from jax.experimental.pallas import tpu_sc as plsc

D = 128
SCALE = math.sqrt(float(D))
LANES = 16
NBUF = 4


@functools.cache
def _build(N, S, V):
    info = plsc.get_sparse_core_info()
    nw = info.num_cores * info.num_subcores
    e_per_w = N // nw
    assert e_per_w % NBUF == 0
    mesh = plsc.VectorSubcoreMesh(core_axis_name="c", subcore_axis_name="s")

    groups = [(k * LANES, 0) for k in range(S // LANES)]
    if S % LANES:
        groups.append((S - LANES, LANES - S % LANES))

    @functools.partial(
        pl.kernel,
        out_type=jax.ShapeDtypeStruct((N, S, D), jnp.float32),
        mesh=mesh,
        scratch_types=[
            pltpu.VMEM((e_per_w, S), jnp.int32),
            [pltpu.VMEM((S, D), jnp.float32) for _ in range(NBUF)],
            [pltpu.SemaphoreType.DMA for _ in range(NBUF)],
            [pltpu.SemaphoreType.DMA for _ in range(NBUF)],
        ],
    )
    def emb_kernel(idx_hbm, table_hbm, out_hbm, idx_v, bufs, gsems, osems):
        wid = lax.axis_index("s") * info.num_cores + lax.axis_index("c")
        base = wid * e_per_w
        pltpu.sync_copy(idx_hbm.at[pl.ds(base, e_per_w)], idx_v)

        def gather(e, slot):
            return pltpu.make_async_copy(
                table_hbm.at[idx_v.at[e]], bufs[slot], gsems[slot]
            )

        def out_copy(e, slot):
            return pltpu.make_async_copy(
                bufs[slot], out_hbm.at[base + e], osems[slot]
            )

        def scale(e, slot):
            rows = bufs[slot]
            for off, skip in groups:
                ivec = idx_v[e, pl.ds(off, LANES)]
                fvec = jnp.where(
                    ivec == 0, jnp.float32(0.0), jnp.float32(SCALE)
                )
                for j in range(skip, LANES):
                    f = fvec[j]
                    for c in range(D // LANES):
                        sl = pl.ds(c * LANES, LANES)
                        rows[off + j, sl] = rows[off + j, sl] * f

        for b in range(NBUF - 1):
            gather(b, b).start()

        @pl.loop(0, e_per_w, step=NBUF)
        def _ring(g):
            for b in range(NBUF):
                e = g + b
                gather(e, b).wait()
                scale(e, b)
                out_copy(e, b).start()
                nxt = e + NBUF - 1
                ns = (b + NBUF - 1) % NBUF
                if b == 0:
                    @pl.when(e == 0)
                    def _():
                        gather(nxt, ns).start()

                    @pl.when(jnp.logical_and(e > 0, nxt < e_per_w))
                    def _():
                        out_copy(e - 1, ns).wait()
                        gather(nxt, ns).start()
                else:
                    @pl.when(nxt < e_per_w)
                    def _():
                        out_copy(e - 1, ns).wait()
                        gather(nxt, ns).start()

        for b in range(NBUF):
            out_copy(e_per_w - NBUF + b, b).wait()

    return emb_kernel


def kernel(inputs, table):
    n, s = inputs.shape
    return _build(n, s, table.shape[0])(inputs, table)


# speedup vs baseline: 9.5260x; 2.0354x over previous
import functools
import math

import jax
import jax.numpy as jnp
from jax import lax
from jax.experimental import pallas as pl
from jax.experimental.pallas import tpu as pltpu
from jax.experimental.pallas import tpu_sc as plsc

D = 128
SCALE = math.sqrt(float(D))
LANES = 16
CHUNK = 80
NBUF = 4


@functools.cache
def _build(B, V):
    info = plsc.get_sparse_core_info()
    nw = info.num_cores * info.num_subcores
    b_per_w = B // nw
    num_chunks = b_per_w // CHUNK
    assert num_chunks % NBUF == 0
    mesh = plsc.VectorSubcoreMesh(core_axis_name="c", subcore_axis_name="s")

    @functools.partial(
        pl.kernel,
        out_type=jax.ShapeDtypeStruct((B, D), jnp.float32),
        mesh=mesh,
        scratch_types=[
            pltpu.VMEM((b_per_w,), jnp.int32),
            [pltpu.VMEM((CHUNK, D), jnp.float32) for _ in range(NBUF)],
            [pltpu.SemaphoreType.DMA for _ in range(NBUF)],
            [pltpu.SemaphoreType.DMA for _ in range(NBUF)],
        ],
    )
    def emb_kernel(idx_hbm, table_hbm, out_hbm, idx_v, bufs, gsems, osems):
        wid = lax.axis_index("s") * info.num_cores + lax.axis_index("c")
        base = wid * b_per_w
        pltpu.sync_copy(idx_hbm.at[pl.ds(base, b_per_w)], idx_v)

        def gather(g, slot):
            return pltpu.make_async_copy(
                table_hbm.at[idx_v.at[pl.ds(g * CHUNK, CHUNK)]],
                bufs[slot],
                gsems[slot],
            )

        def out_copy(g, slot):
            return pltpu.make_async_copy(
                bufs[slot],
                out_hbm.at[pl.ds(base + g * CHUNK, CHUNK)],
                osems[slot],
            )

        def scale(g, slot):
            rows = bufs[slot]
            off = g * CHUNK

            @pl.loop(0, CHUNK // LANES)
            def _grp(k):
                ivec = idx_v[pl.ds(off + k * LANES, LANES)]
                fvec = jnp.where(
                    ivec == 0, jnp.float32(0.0), jnp.float32(SCALE)
                )
                for j in range(LANES):
                    f = fvec[j]
                    for c in range(D // LANES):
                        sl = pl.ds(c * LANES, LANES)
                        rows[k * LANES + j, sl] = rows[k * LANES + j, sl] * f

        for b in range(NBUF - 1):
            gather(b, b).start()

        @pl.loop(0, num_chunks, step=NBUF)
        def _ring(g):
            for b in range(NBUF):
                gc = g + b
                gather(gc, b).wait()
                scale(gc, b)
                out_copy(gc, b).start()
                nxt = gc + NBUF - 1
                ns = (b + NBUF - 1) % NBUF
                if b == 0:
                    @pl.when(gc == 0)
                    def _():
                        gather(nxt, ns).start()

                    @pl.when(jnp.logical_and(gc > 0, nxt < num_chunks))
                    def _():
                        out_copy(gc - 1, ns).wait()
                        gather(nxt, ns).start()
                else:
                    @pl.when(nxt < num_chunks)
                    def _():
                        out_copy(gc - 1, ns).wait()
                        gather(nxt, ns).start()

        for b in range(NBUF):
            out_copy(num_chunks - NBUF + b, b).wait()

    return emb_kernel


def kernel(inputs, table):
    n, s = inputs.shape
    idx = inputs.T.reshape(-1)
    out = _build(n * s, table.shape[0])(idx, table)
    return out.reshape(s, n, D).transpose(1, 0, 2)


# speedup vs baseline: 9.7391x; 1.0224x over previous
import functools
import math

import jax
import jax.numpy as jnp
from jax import lax
from jax.experimental import pallas as pl
from jax.experimental.pallas import tpu as pltpu
from jax.experimental.pallas import tpu_sc as plsc

D = 128
SCALE = math.sqrt(float(D))
LANES = 16
CHUNK = 128
NBUF = 5


@functools.cache
def _build(N, S, V):
    info = plsc.get_sparse_core_info()
    nw = info.num_cores * info.num_subcores
    assert N % (nw * CHUNK) == 0 and S % NBUF == 0
    mesh = plsc.VectorSubcoreMesh(core_axis_name="c", subcore_axis_name="s")

    @functools.partial(
        pl.kernel,
        out_type=jax.ShapeDtypeStruct((N * S, D), jnp.float32),
        mesh=mesh,
        scratch_types=[
            pltpu.VMEM((S, CHUNK), jnp.int32),
            [pltpu.VMEM((CHUNK, D), jnp.float32) for _ in range(NBUF)],
            [pltpu.SemaphoreType.DMA for _ in range(NBUF)],
            [pltpu.SemaphoreType.DMA for _ in range(NBUF)],
        ],
    )
    def emb_kernel(idx_hbm, table_hbm, out_hbm, idx_v, bufs, gsems, osems):
        wid = lax.axis_index("s") * info.num_cores + lax.axis_index("c")
        col0 = wid * CHUNK
        pltpu.sync_copy(idx_hbm.at[:, pl.ds(col0, CHUNK)], idx_v)

        def gather(s, slot):
            return pltpu.make_async_copy(
                table_hbm.at[idx_v.at[s]], bufs[slot], gsems[slot]
            )

        def out_copy(s, slot):
            return pltpu.make_async_copy(
                bufs[slot],
                out_hbm.at[pl.ds(s * N + col0, CHUNK)],
                osems[slot],
            )

        def scale(s, slot):
            rows = bufs[slot]

            @pl.loop(0, CHUNK // LANES)
            def _grp(k):
                ivec = idx_v[s, pl.ds(k * LANES, LANES)]
                fvec = jnp.where(
                    ivec == 0, jnp.float32(0.0), jnp.float32(SCALE)
                )
                for j in range(LANES):
                    f = fvec[j]
                    for c in range(D // LANES):
                        sl = pl.ds(c * LANES, LANES)
                        rows[k * LANES + j, sl] = rows[k * LANES + j, sl] * f

        for b in range(NBUF - 1):
            gather(b, b).start()

        @pl.loop(0, S, step=NBUF)
        def _ring(g):
            for b in range(NBUF):
                sc = g + b
                gather(sc, b).wait()
                scale(sc, b)
                out_copy(sc, b).start()
                nxt = sc + NBUF - 1
                ns = (b + NBUF - 1) % NBUF
                if b == 0:
                    @pl.when(sc == 0)
                    def _():
                        gather(nxt, ns).start()

                    @pl.when(jnp.logical_and(sc > 0, nxt < S))
                    def _():
                        out_copy(sc - 1, ns).wait()
                        gather(nxt, ns).start()
                else:
                    @pl.when(nxt < S)
                    def _():
                        out_copy(sc - 1, ns).wait()
                        gather(nxt, ns).start()

        for b in range(NBUF):
            out_copy(S - NBUF + b, b).wait()

    return emb_kernel


def kernel(inputs, table):
    n, s = inputs.shape
    out = _build(n, s, table.shape[0])(inputs.T, table)
    return out.reshape(s, n, D).transpose(1, 0, 2)
